# baseline (speedup 1.0000x reference)
import functools

import jax
import jax.numpy as jnp
import numpy as np
from jax import lax
from jax.experimental import pallas as pl
from jax.experimental.pallas import tpu as pltpu
from jax.experimental.pallas import tpu_sc as plsc

_M = 65536
_D = 256
_LANES = 16
_VECS = _D // _LANES
_NC = 2
_NS = 16
_NW = _NC * _NS
_ROWS_PER_W = _M // _NW
_CHUNK = 64
_CHUNKS_PER_W = _ROWS_PER_W // _CHUNK
_ROUND_C = np.float32(12582912.0)

_TC_BLOCK = 8192


def _quantize_chunk(ibuf, obuf):
    @plsc.parallel_loop(0, _CHUNK, 1, unroll=8)
    def row_body(r):
        vecs = [ibuf[r, pl.ds(j * _LANES, _LANES)] for j in range(_VECS)]
        m = jnp.abs(vecs[0])
        for j in range(1, _VECS):
            m = jnp.maximum(m, jnp.abs(vecs[j]))
        mv = lax.broadcast_in_dim(jnp.max(m), (_LANES,), ())
        nonzero = mv > 0.0
        inv = jnp.where(nonzero, 7.0 / mv, np.float32(1.0))
        scale = jnp.where(nonzero, mv * np.float32(1.0 / 7.0), np.float32(1.0))
        for j in range(_VECS):
            y = vecs[j] * inv
            q = (y + _ROUND_C) - _ROUND_C
            obuf[r, pl.ds(j * _LANES, _LANES)] = q * scale


def _sc_body(x_hbm, out_hbm, ibuf, obuf, isems, osems):
    wid = lax.axis_index("s") * _NC + lax.axis_index("c")
    base = wid * _ROWS_PER_W

    def in_copy(c, b):
        return pltpu.make_async_copy(
            x_hbm.at[pl.ds(base + c * _CHUNK, _CHUNK)], ibuf.at[b], isems.at[b])

    def out_copy(c, b):
        return pltpu.make_async_copy(
            obuf.at[b], out_hbm.at[pl.ds(base + c * _CHUNK, _CHUNK)], osems.at[b])

    in_copy(0, 0).start()

    def pair_body(i, carry):
        c0 = 2 * i
        in_copy(c0 + 1, 1).start()
        in_copy(c0, 0).wait()

        @pl.when(i > 0)
        def _():
            out_copy(c0, 0).wait()

        _quantize_chunk(ibuf.at[0], obuf.at[0])
        out_copy(c0, 0).start()

        @pl.when(c0 + 2 < _CHUNKS_PER_W)
        def _():
            in_copy(c0 + 2, 0).start()

        in_copy(c0 + 1, 1).wait()

        @pl.when(i > 0)
        def _():
            out_copy(c0 + 1, 1).wait()

        _quantize_chunk(ibuf.at[1], obuf.at[1])
        out_copy(c0 + 1, 1).start()
        return carry

    lax.fori_loop(0, _CHUNKS_PER_W // 2, pair_body, 0)
    out_copy(_CHUNKS_PER_W - 2, 0).wait()
    out_copy(_CHUNKS_PER_W - 1, 1).wait()


def _sc_quantize(x):
    out = jax.ShapeDtypeStruct((_M, _D), jnp.float32)
    mesh = plsc.VectorSubcoreMesh(core_axis_name="c", subcore_axis_name="s")
    f = functools.partial(
        pl.kernel,
        out_type=out,
        mesh=mesh,
        compiler_params=pltpu.CompilerParams(needs_layout_passes=False),
        scratch_types=[
            pltpu.VMEM((2, _CHUNK, _D), jnp.float32),
            pltpu.VMEM((2, _CHUNK, _D), jnp.float32),
            pltpu.SemaphoreType.DMA((2,)),
            pltpu.SemaphoreType.DMA((2,)),
        ],
    )(_sc_body)
    return f(x)


def _tc_body(x_ref, o_ref):
    x = x_ref[...]
    maxabs = jnp.max(jnp.abs(x), axis=1, keepdims=True)
    scale = jnp.where(maxabs > 0, maxabs * np.float32(1.0 / 7.0),
                      np.float32(1.0))
    inv = jnp.where(maxabs > 0, 7.0 / maxabs, np.float32(1.0))
    q = (x * inv + _ROUND_C) - _ROUND_C
    o_ref[...] = q * scale


def _tc_quantize(x):
    return pl.pallas_call(
        _tc_body,
        grid=(_M // _TC_BLOCK,),
        in_specs=[pl.BlockSpec((_TC_BLOCK, _D), lambda i: (i, 0))],
        out_specs=pl.BlockSpec((_TC_BLOCK, _D), lambda i: (i, 0)),
        out_shape=jax.ShapeDtypeStruct((_M, _D), jnp.float32),
        compiler_params=pltpu.CompilerParams(
            vmem_limit_bytes=128 * 1024 * 1024),
    )(x)


@jax.jit
def kernel(memory_mean, memory_logvar):
    logvar_deq = _sc_quantize(memory_logvar)
    mean_deq = _tc_quantize(memory_mean)
    return mean_deq, logvar_deq


# speedup vs baseline: 1.0232x; 1.0232x over previous
import functools

import jax
import jax.numpy as jnp
import numpy as np
from jax import lax
from jax.experimental import pallas as pl
from jax.experimental.pallas import tpu as pltpu
from jax.experimental.pallas import tpu_sc as plsc

_M = 65536
_D = 256
_LANES = 16
_VECS = _D // _LANES
_NC = 2
_NS = 16
_NW = _NC * _NS
_ROWS_PER_W = _M // _NW
_CHUNK = 64
_CHUNKS_PER_W = _ROWS_PER_W // _CHUNK
_ROUND_C = np.float32(12582912.0)

_TC_BLOCK = 2048


def _quantize_chunk(ibuf, obuf):
    @plsc.parallel_loop(0, _CHUNK, 1, unroll=8)
    def row_body(r):
        vecs = [ibuf[r, pl.ds(j * _LANES, _LANES)] for j in range(_VECS)]
        m = jnp.abs(vecs[0])
        for j in range(1, _VECS):
            m = jnp.maximum(m, jnp.abs(vecs[j]))
        mv = lax.broadcast_in_dim(jnp.max(m), (_LANES,), ())
        nonzero = mv > 0.0
        inv = jnp.where(nonzero, 7.0 / mv, np.float32(1.0))
        scale = jnp.where(nonzero, mv * np.float32(1.0 / 7.0), np.float32(1.0))
        for j in range(_VECS):
            y = vecs[j] * inv
            q = (y + _ROUND_C) - _ROUND_C
            obuf[r, pl.ds(j * _LANES, _LANES)] = q * scale


def _sc_body(x_hbm, out_hbm, ibuf, obuf, isems, osems):
    wid = lax.axis_index("s") * _NC + lax.axis_index("c")
    base = wid * _ROWS_PER_W

    def in_copy(c, b):
        return pltpu.make_async_copy(
            x_hbm.at[pl.ds(base + c * _CHUNK, _CHUNK)], ibuf.at[b], isems.at[b])

    def out_copy(c, b):
        return pltpu.make_async_copy(
            obuf.at[b], out_hbm.at[pl.ds(base + c * _CHUNK, _CHUNK)], osems.at[b])

    in_copy(0, 0).start()

    def pair_body(i, carry):
        c0 = 2 * i
        in_copy(c0 + 1, 1).start()
        in_copy(c0, 0).wait()

        @pl.when(i > 0)
        def _():
            out_copy(c0, 0).wait()

        _quantize_chunk(ibuf.at[0], obuf.at[0])
        out_copy(c0, 0).start()

        @pl.when(c0 + 2 < _CHUNKS_PER_W)
        def _():
            in_copy(c0 + 2, 0).start()

        in_copy(c0 + 1, 1).wait()

        @pl.when(i > 0)
        def _():
            out_copy(c0 + 1, 1).wait()

        _quantize_chunk(ibuf.at[1], obuf.at[1])
        out_copy(c0 + 1, 1).start()
        return carry

    lax.fori_loop(0, _CHUNKS_PER_W // 2, pair_body, 0)
    out_copy(_CHUNKS_PER_W - 2, 0).wait()
    out_copy(_CHUNKS_PER_W - 1, 1).wait()


def _sc_quantize(x):
    out = jax.ShapeDtypeStruct((_M, _D), jnp.float32)
    mesh = plsc.VectorSubcoreMesh(core_axis_name="c", subcore_axis_name="s")
    f = functools.partial(
        pl.kernel,
        out_type=out,
        mesh=mesh,
        compiler_params=pltpu.CompilerParams(
            needs_layout_passes=False, skip_device_barrier=True),
        scratch_types=[
            pltpu.VMEM((2, _CHUNK, _D), jnp.float32),
            pltpu.VMEM((2, _CHUNK, _D), jnp.float32),
            pltpu.SemaphoreType.DMA((2,)),
            pltpu.SemaphoreType.DMA((2,)),
        ],
    )(_sc_body)
    return f(x)


def _tc_body(x_ref, o_ref):
    x = x_ref[...]
    maxabs = jnp.max(jnp.abs(x), axis=1, keepdims=True)
    scale = jnp.where(maxabs > 0, maxabs * np.float32(1.0 / 7.0),
                      np.float32(1.0))
    inv = jnp.where(maxabs > 0, 7.0 / maxabs, np.float32(1.0))
    q = (x * inv + _ROUND_C) - _ROUND_C
    o_ref[...] = q * scale


def _tc_quantize(x):
    return pl.pallas_call(
        _tc_body,
        grid=(_M // _TC_BLOCK,),
        in_specs=[pl.BlockSpec((_TC_BLOCK, _D), lambda i: (i, 0))],
        out_specs=pl.BlockSpec((_TC_BLOCK, _D), lambda i: (i, 0)),
        out_shape=jax.ShapeDtypeStruct((_M, _D), jnp.float32),
        compiler_params=pltpu.CompilerParams(
            vmem_limit_bytes=128 * 1024 * 1024),
    )(x)


@jax.jit
def kernel(memory_mean, memory_logvar):
    logvar_deq = _sc_quantize(memory_logvar)
    mean_deq = _tc_quantize(memory_mean)
    return mean_deq, logvar_deq


# speedup vs baseline: 1.0258x; 1.0025x over previous
import functools

import jax
import jax.numpy as jnp
import numpy as np
from jax import lax
from jax.experimental import pallas as pl
from jax.experimental.pallas import tpu as pltpu
from jax.experimental.pallas import tpu_sc as plsc

_M = 65536
_D = 256
_LANES = 16
_VECS = _D // _LANES
_NC = 2
_NS = 16
_NW = _NC * _NS
_ROWS_PER_W = _M // _NW
_CHUNK = 64
_CHUNKS_PER_W = _ROWS_PER_W // _CHUNK
_ROUND_C = np.float32(12582912.0)

_TC_BLOCK = 2048


def _quantize_chunk(ibuf, obuf):
    @plsc.parallel_loop(0, _CHUNK, 1, unroll=8)
    def row_body(r):
        vecs = [ibuf[r, pl.ds(j * _LANES, _LANES)] for j in range(_VECS)]
        m = jnp.abs(vecs[0])
        for j in range(1, _VECS):
            m = jnp.maximum(m, jnp.abs(vecs[j]))
        mv = lax.broadcast_in_dim(jnp.max(m), (_LANES,), ())
        nonzero = mv > 0.0
        inv = jnp.where(nonzero, 7.0 / mv, np.float32(1.0))
        scale = jnp.where(nonzero, mv * np.float32(1.0 / 7.0), np.float32(1.0))
        for j in range(_VECS):
            y = vecs[j] * inv
            q = (y + _ROUND_C) - _ROUND_C
            obuf[r, pl.ds(j * _LANES, _LANES)] = q * scale


def _sc_body(x_hbm, out_hbm, ibuf, obuf, isems, osems):
    wid = lax.axis_index("s") * _NC + lax.axis_index("c")
    base = wid * _ROWS_PER_W

    def in_copy(c, b):
        return pltpu.make_async_copy(
            x_hbm.at[pl.ds(base + c * _CHUNK, _CHUNK)], ibuf.at[b], isems.at[b])

    def out_copy(c, b):
        return pltpu.make_async_copy(
            obuf.at[b], out_hbm.at[pl.ds(base + c * _CHUNK, _CHUNK)], osems.at[b])

    in_copy(0, 0).start()

    def pair_body(i, carry):
        c0 = 2 * i
        in_copy(c0 + 1, 1).start()
        in_copy(c0, 0).wait()

        @pl.when(i > 0)
        def _():
            out_copy(c0, 0).wait()

        _quantize_chunk(ibuf.at[0], obuf.at[0])
        out_copy(c0, 0).start()

        @pl.when(c0 + 2 < _CHUNKS_PER_W)
        def _():
            in_copy(c0 + 2, 0).start()

        in_copy(c0 + 1, 1).wait()

        @pl.when(i > 0)
        def _():
            out_copy(c0 + 1, 1).wait()

        _quantize_chunk(ibuf.at[1], obuf.at[1])
        out_copy(c0 + 1, 1).start()
        return carry

    lax.fori_loop(0, _CHUNKS_PER_W // 2, pair_body, 0)
    out_copy(_CHUNKS_PER_W - 2, 0).wait()
    out_copy(_CHUNKS_PER_W - 1, 1).wait()


def _sc_quantize(x):
    out = jax.ShapeDtypeStruct((_M, _D), jnp.float32)
    mesh = plsc.VectorSubcoreMesh(core_axis_name="c", subcore_axis_name="s")
    f = functools.partial(
        pl.kernel,
        out_type=out,
        mesh=mesh,
        compiler_params=pltpu.CompilerParams(needs_layout_passes=False),
        scratch_types=[
            pltpu.VMEM((2, _CHUNK, _D), jnp.float32),
            pltpu.VMEM((2, _CHUNK, _D), jnp.float32),
            pltpu.SemaphoreType.DMA((2,)),
            pltpu.SemaphoreType.DMA((2,)),
        ],
    )(_sc_body)
    return f(x)


def _tc_body(x_ref, o_ref):
    x = x_ref[...]
    maxabs = jnp.max(jnp.abs(x), axis=1, keepdims=True)
    scale = jnp.where(maxabs > 0, maxabs * np.float32(1.0 / 7.0),
                      np.float32(1.0))
    inv = jnp.where(maxabs > 0, 7.0 / maxabs, np.float32(1.0))
    q = (x * inv + _ROUND_C) - _ROUND_C
    o_ref[...] = q * scale


def _tc_quantize(x):
    return pl.pallas_call(
        _tc_body,
        grid=(_M // _TC_BLOCK,),
        in_specs=[pl.BlockSpec((_TC_BLOCK, _D), lambda i: (i, 0))],
        out_specs=pl.BlockSpec((_TC_BLOCK, _D), lambda i: (i, 0)),
        out_shape=jax.ShapeDtypeStruct((_M, _D), jnp.float32),
        compiler_params=pltpu.CompilerParams(
            vmem_limit_bytes=128 * 1024 * 1024),
    )(x)


@jax.jit
def kernel(memory_mean, memory_logvar):
    logvar_deq = _sc_quantize(memory_logvar)
    mean_deq = _tc_quantize(memory_mean)
    return mean_deq, logvar_deq
